# baseline (speedup 1.0000x reference)
import functools

import jax
import jax.numpy as jnp
from jax import lax
from jax.experimental import pallas as pl
from jax.experimental.pallas import tpu as pltpu
from jax.experimental.pallas import tpu_sc as plsc

N_NODES = 10000
N_EDGES = 320000
D = 128
DH = D // 2

NC = 2
NS = 16

EPT = N_EDGES // NS
CHUNK = 125
NCHUNK = EPT // CHUNK
NQ = NCHUNK // 4

N_PAD = 10240
ROWS_PER_SUB = N_PAD // NS
ZROWS = 128
NZCOPY = ROWS_PER_SUB // ZROWS


def _sc_scatter(nh2, src_idx, dst_idx):
  mesh = plsc.VectorSubcoreMesh(core_axis_name="c", subcore_axis_name="s")

  @functools.partial(
      pl.kernel,
      out_type=jax.ShapeDtypeStruct((NC, N_PAD, DH), jnp.float32),
      name="gin_segment_sum",
      mesh=mesh,
      scratch_types=[
          pltpu.VMEM((NCHUNK, CHUNK), jnp.int32),
          pltpu.VMEM((NCHUNK, CHUNK), jnp.int32),
          [pltpu.VMEM((CHUNK, DH), jnp.float32)] * 4,
          pltpu.VMEM((ZROWS, DH), jnp.float32),
          pltpu.VMEM_SHARED((N_PAD, DH), jnp.float32),
          [pltpu.SemaphoreType.DMA] * 4,
          [pltpu.SemaphoreType.DMA] * 4,
      ],
      compiler_params=pltpu.CompilerParams(use_tc_tiling_on_sc=False),
  )
  def k(nh_hbm, src_hbm, dst_hbm, out_hbm, src_v, dst_v, rows, stage_v,
        acc, gsem, ssem):
    cid = lax.axis_index("c")
    sid = lax.axis_index("s")
    base = sid * ROWS_PER_SUB

    zeros16 = jnp.zeros((16,), jnp.float32)

    def zrow(i, carry):
      for j in range(DH // 16):
        stage_v[i, pl.ds(j * 16, 16)] = zeros16
      return carry

    lax.fori_loop(0, ZROWS, zrow, 0)

    def zcopy(c, carry):
      pltpu.sync_copy(stage_v, acc.at[pl.ds(base + c * ZROWS, ZROWS)])
      return carry

    lax.fori_loop(0, NZCOPY, zcopy, 0)
    plsc.subcore_barrier()

    pltpu.sync_copy(src_hbm.at[sid], src_v)
    pltpu.sync_copy(dst_hbm.at[sid], dst_v)

    table = nh_hbm.at[pl.ds(cid, 2 * N_NODES - 1)]

    def gather(j, b):
      return pltpu.async_copy(table.at[src_v.at[j]], rows[b], gsem[b])

    def scatter(j, b):
      return pltpu.async_copy(rows[b], acc.at[dst_v.at[j]], ssem[b], add=True)

    def wait_gather(j, b):
      pltpu.make_async_copy(table.at[src_v.at[j]], rows[b], gsem[b]).wait()

    def wait_scatter(j, b):
      pltpu.make_async_copy(rows[b], acc.at[dst_v.at[j]], ssem[b]).wait()

    gather(0, 0)
    gather(1, 1)

    def body(g, carry):
      j0 = 4 * g
      for b in range(4):
        j = j0 + b
        p = (b + 2) % 4
        wait_gather(j, b)
        scatter(j, b)
        if b < 2:
          @pl.when(g >= 1)
          def _():
            wait_scatter(j - 2, p)

          gather(j + 2, p)
        else:
          wait_scatter(j - 2, p)

          @pl.when(g < NQ - 1)
          def _():
            gather(j + 2, p)
      return carry

    lax.fori_loop(0, NQ, body, 0)
    wait_scatter(NCHUNK - 2, 2)
    wait_scatter(NCHUNK - 1, 3)
    plsc.subcore_barrier()

    def ocopy(c, carry):
      pltpu.sync_copy(acc.at[pl.ds(base + c * ZROWS, ZROWS)], stage_v)
      pltpu.sync_copy(stage_v, out_hbm.at[cid, pl.ds(base + c * ZROWS, ZROWS)])
      return carry

    lax.fori_loop(0, NZCOPY, ocopy, 0)

  return k(nh2, src_idx, dst_idx)


def _tc_mlp(nh, parts, W1, b1, W2, b2, eps):
  BLK = 1000
  grid = (N_NODES // BLK,)

  def body(eps_ref, nh_ref, p_ref, w1_ref, b1_ref, w2_ref, b2_ref, out_ref):
    scale = 1.0 + eps_ref[0]
    nz = jnp.concatenate([p_ref[0], p_ref[1]], axis=-1)
    x = scale * nh_ref[...] + nz
    h = jnp.maximum(
        jnp.dot(x, w1_ref[...], preferred_element_type=jnp.float32)
        + b1_ref[...], 0.0)
    out_ref[...] = (
        jnp.dot(h, w2_ref[...], preferred_element_type=jnp.float32)
        + b2_ref[...])

  return pl.pallas_call(
      body,
      grid=grid,
      in_specs=[
          pl.BlockSpec(memory_space=pltpu.SMEM),
          pl.BlockSpec((BLK, D), lambda i: (i, 0)),
          pl.BlockSpec((NC, BLK, DH), lambda i: (0, i, 0)),
          pl.BlockSpec((D, D), lambda i: (0, 0)),
          pl.BlockSpec((1, D), lambda i: (0, 0)),
          pl.BlockSpec((D, D), lambda i: (0, 0)),
          pl.BlockSpec((1, D), lambda i: (0, 0)),
      ],
      out_specs=pl.BlockSpec((BLK, D), lambda i: (i, 0)),
      out_shape=jax.ShapeDtypeStruct((N_NODES, D), jnp.float32),
  )(eps, nh, parts, W1, b1.reshape(1, D), W2, b2.reshape(1, D))


@jax.jit
def _gin(nh, edge_index, W1, b1, W2, b2, eps):
  ei = edge_index.astype(jnp.int32)
  src2 = (ei[0] * 2).reshape(NS, NCHUNK, CHUNK)
  dst = ei[1].reshape(NS, NCHUNK, CHUNK)
  nh2 = nh.reshape(2 * N_NODES, DH)
  parts = _sc_scatter(nh2, src2, dst)
  return _tc_mlp(nh, parts, W1, b1, W2, b2, eps)


def kernel(nh, eh, edge_index, W1, b1, W2, b2, eps):
  return (_gin(nh, edge_index, W1, b1, W2, b2, eps), eh)


# speedup vs baseline: 1.0632x; 1.0632x over previous
import functools

import jax
import jax.numpy as jnp
from jax import lax
from jax.experimental import pallas as pl
from jax.experimental.pallas import tpu as pltpu
from jax.experimental.pallas import tpu_sc as plsc

N_NODES = 10000
N_EDGES = 320000
D = 128
DH = D // 2

NC = 2
NS = 16

EPT = N_EDGES // NS
CHUNK = 125
NCHUNK = EPT // CHUNK
NQ = NCHUNK // 4

N_PAD = 10240
ROWS_PER_SUB = N_PAD // NS
ZROWS = 128
NZCOPY = ROWS_PER_SUB // ZROWS


def _sc_scatter(nh2, src_idx, dst_idx):
  mesh = plsc.VectorSubcoreMesh(core_axis_name="c", subcore_axis_name="s")

  @functools.partial(
      pl.kernel,
      out_type=jax.ShapeDtypeStruct((N_PAD, D), jnp.float32),
      name="gin_segment_sum",
      mesh=mesh,
      scratch_types=[
          pltpu.VMEM((NCHUNK, CHUNK), jnp.int32),
          pltpu.VMEM((NCHUNK, CHUNK), jnp.int32),
          [pltpu.VMEM((CHUNK, DH), jnp.float32)] * 4,
          pltpu.VMEM((ZROWS, DH), jnp.float32),
          pltpu.VMEM_SHARED((N_PAD, DH), jnp.float32),
          [pltpu.SemaphoreType.DMA] * 4,
          [pltpu.SemaphoreType.DMA] * 4,
      ],
      compiler_params=pltpu.CompilerParams(use_tc_tiling_on_sc=False),
  )
  def k(nh_hbm, src_hbm, dst_hbm, out_hbm, src_v, dst_v, rows, stage_v,
        acc, gsem, ssem):
    cid = lax.axis_index("c")
    sid = lax.axis_index("s")
    base = sid * ROWS_PER_SUB

    zeros16 = jnp.zeros((16,), jnp.float32)

    def zrow(i, carry):
      for j in range(DH // 16):
        stage_v[i, pl.ds(j * 16, 16)] = zeros16
      return carry

    lax.fori_loop(0, ZROWS, zrow, 0)

    def zcopy(c, carry):
      pltpu.sync_copy(stage_v, acc.at[pl.ds(base + c * ZROWS, ZROWS)])
      return carry

    lax.fori_loop(0, NZCOPY, zcopy, 0)
    plsc.subcore_barrier()

    pltpu.sync_copy(src_hbm.at[sid], src_v)
    pltpu.sync_copy(dst_hbm.at[sid], dst_v)

    table = nh_hbm.at[pl.ds(cid, 2 * N_NODES - 1)]

    def gather(j, b):
      return pltpu.async_copy(table.at[src_v.at[j]], rows[b], gsem[b])

    def scatter(j, b):
      return pltpu.async_copy(rows[b], acc.at[dst_v.at[j]], ssem[b], add=True)

    def wait_gather(j, b):
      pltpu.make_async_copy(table.at[src_v.at[j]], rows[b], gsem[b]).wait()

    def wait_scatter(j, b):
      pltpu.make_async_copy(rows[b], acc.at[dst_v.at[j]], ssem[b]).wait()

    gather(0, 0)
    gather(1, 1)

    def body(g, carry):
      j0 = 4 * g
      for b in range(4):
        j = j0 + b
        p = (b + 2) % 4
        wait_gather(j, b)
        scatter(j, b)
        if b < 2:
          @pl.when(g >= 1)
          def _():
            wait_scatter(j - 2, p)

          gather(j + 2, p)
        else:
          wait_scatter(j - 2, p)

          @pl.when(g < NQ - 1)
          def _():
            gather(j + 2, p)
      return carry

    lax.fori_loop(0, NQ, body, 0)
    wait_scatter(NCHUNK - 2, 2)
    wait_scatter(NCHUNK - 1, 3)
    plsc.subcore_barrier()

    def ocopy(c, carry):
      pltpu.sync_copy(acc.at[pl.ds(base + c * ZROWS, ZROWS)], stage_v)
      pltpu.sync_copy(
          stage_v,
          out_hbm.at[pl.ds(base + c * ZROWS, ZROWS), pl.ds(cid * DH, DH)])
      return carry

    lax.fori_loop(0, NZCOPY, ocopy, 0)

  return k(nh2, src_idx, dst_idx)


def _tc_mlp(nh, parts, W1, b1, W2, b2, eps):
  BLK = 1000
  grid = (N_NODES // BLK,)

  def body(eps_ref, nh_ref, p_ref, w1_ref, b1_ref, w2_ref, b2_ref, out_ref):
    scale = 1.0 + eps_ref[0]
    x = scale * nh_ref[...] + p_ref[...]
    h = jnp.maximum(
        jnp.dot(x, w1_ref[...], preferred_element_type=jnp.float32)
        + b1_ref[...], 0.0)
    out_ref[...] = (
        jnp.dot(h, w2_ref[...], preferred_element_type=jnp.float32)
        + b2_ref[...])

  return pl.pallas_call(
      body,
      grid=grid,
      in_specs=[
          pl.BlockSpec(memory_space=pltpu.SMEM),
          pl.BlockSpec((BLK, D), lambda i: (i, 0)),
          pl.BlockSpec((BLK, D), lambda i: (i, 0)),
          pl.BlockSpec((D, D), lambda i: (0, 0)),
          pl.BlockSpec((1, D), lambda i: (0, 0)),
          pl.BlockSpec((D, D), lambda i: (0, 0)),
          pl.BlockSpec((1, D), lambda i: (0, 0)),
      ],
      out_specs=pl.BlockSpec((BLK, D), lambda i: (i, 0)),
      out_shape=jax.ShapeDtypeStruct((N_NODES, D), jnp.float32),
  )(eps, nh, parts, W1, b1.reshape(1, D), W2, b2.reshape(1, D))


@jax.jit
def _gin(nh, edge_index, W1, b1, W2, b2, eps):
  ei = edge_index.astype(jnp.int32)
  src2 = (ei[0] * 2).reshape(NS, NCHUNK, CHUNK)
  dst = ei[1].reshape(NS, NCHUNK, CHUNK)
  nh2 = nh.reshape(2 * N_NODES, DH)
  parts = _sc_scatter(nh2, src2, dst)
  return _tc_mlp(nh, parts, W1, b1, W2, b2, eps)


def kernel(nh, eh, edge_index, W1, b1, W2, b2, eps):
  return (_gin(nh, edge_index, W1, b1, W2, b2, eps), eh)


# speedup vs baseline: 1.0787x; 1.0146x over previous
import functools

import jax
import jax.numpy as jnp
from jax import lax
from jax.experimental import pallas as pl
from jax.experimental.pallas import tpu as pltpu
from jax.experimental.pallas import tpu_sc as plsc

N_NODES = 10000
N_EDGES = 320000
D = 128
DH = D // 2

NC = 2
NS = 16

EPT = N_EDGES // NS
CHUNK = 125
NCHUNK = EPT // CHUNK
NQ = NCHUNK // 4

N_PAD = 10240
ROWS_PER_SUB = N_PAD // NS
ZROWS = 128
NZCOPY = ROWS_PER_SUB // ZROWS


def _sc_scatter(nh2, src_idx, dst_idx):
  mesh = plsc.VectorSubcoreMesh(core_axis_name="c", subcore_axis_name="s")

  @functools.partial(
      pl.kernel,
      out_type=jax.ShapeDtypeStruct((N_PAD, D), jnp.float32),
      name="gin_segment_sum",
      mesh=mesh,
      scratch_types=[
          pltpu.VMEM((NCHUNK, CHUNK), jnp.int32),
          pltpu.VMEM((NCHUNK, CHUNK), jnp.int32),
          [pltpu.VMEM((CHUNK, DH), jnp.float32)] * 4,
          pltpu.VMEM((ZROWS, DH), jnp.float32),
          pltpu.VMEM_SHARED((N_PAD, DH), jnp.float32),
          [pltpu.SemaphoreType.DMA] * 4,
          [pltpu.SemaphoreType.DMA] * 4,
      ],
      compiler_params=pltpu.CompilerParams(use_tc_tiling_on_sc=False),
  )
  def k(nh_hbm, src_hbm, dst_hbm, out_hbm, src_v, dst_v, rows, stage_v,
        acc, gsem, ssem):
    cid = lax.axis_index("c")
    sid = lax.axis_index("s")
    base = sid * ROWS_PER_SUB

    idx_a = pltpu.async_copy(src_hbm.at[sid], src_v, gsem[0])
    idx_b = pltpu.async_copy(dst_hbm.at[sid], dst_v, gsem[1])

    zeros16 = jnp.zeros((16,), jnp.float32)

    def zrow(i, carry):
      for j in range(DH // 16):
        stage_v[i, pl.ds(j * 16, 16)] = zeros16
      return carry

    lax.fori_loop(0, ZROWS, zrow, 0)

    def zcopy(c, carry):
      pltpu.sync_copy(stage_v, acc.at[pl.ds(base + c * ZROWS, ZROWS)])
      return carry

    lax.fori_loop(0, NZCOPY, zcopy, 0)
    idx_a.wait()
    idx_b.wait()
    plsc.subcore_barrier()

    table = nh_hbm.at[pl.ds(cid, 2 * N_NODES - 1)]

    def gather(j, b):
      return pltpu.async_copy(table.at[src_v.at[j]], rows[b], gsem[b])

    def scatter(j, b):
      return pltpu.async_copy(rows[b], acc.at[dst_v.at[j]], ssem[b], add=True)

    def wait_gather(j, b):
      pltpu.make_async_copy(table.at[src_v.at[j]], rows[b], gsem[b]).wait()

    def wait_scatter(j, b):
      pltpu.make_async_copy(rows[b], acc.at[dst_v.at[j]], ssem[b]).wait()

    gather(0, 0)
    gather(1, 1)

    def body(g, carry):
      j0 = 4 * g
      for b in range(4):
        j = j0 + b
        p = (b + 2) % 4
        wait_gather(j, b)
        scatter(j, b)
        if b < 2:
          @pl.when(g >= 1)
          def _():
            wait_scatter(j - 2, p)

          gather(j + 2, p)
        else:
          wait_scatter(j - 2, p)

          @pl.when(g < NQ - 1)
          def _():
            gather(j + 2, p)
      return carry

    lax.fori_loop(0, NQ, body, 0)
    wait_scatter(NCHUNK - 2, 2)
    wait_scatter(NCHUNK - 1, 3)
    plsc.subcore_barrier()

    def ocopy(c, carry):
      pltpu.sync_copy(acc.at[pl.ds(base + c * ZROWS, ZROWS)], stage_v)
      pltpu.sync_copy(
          stage_v,
          out_hbm.at[pl.ds(base + c * ZROWS, ZROWS), pl.ds(cid * DH, DH)])
      return carry

    lax.fori_loop(0, NZCOPY, ocopy, 0)

  return k(nh2, src_idx, dst_idx)


def _tc_mlp(nh, parts, W1, b1, W2, b2, eps):
  BLK = 1000
  grid = (N_NODES // BLK,)

  def body(eps_ref, nh_ref, p_ref, w1_ref, b1_ref, w2_ref, b2_ref, out_ref):
    scale = 1.0 + eps_ref[0]
    x = scale * nh_ref[...] + p_ref[...]
    h = jnp.maximum(
        jnp.dot(x, w1_ref[...], preferred_element_type=jnp.float32)
        + b1_ref[...], 0.0)
    out_ref[...] = (
        jnp.dot(h, w2_ref[...], preferred_element_type=jnp.float32)
        + b2_ref[...])

  return pl.pallas_call(
      body,
      grid=grid,
      in_specs=[
          pl.BlockSpec(memory_space=pltpu.SMEM),
          pl.BlockSpec((BLK, D), lambda i: (i, 0)),
          pl.BlockSpec((BLK, D), lambda i: (i, 0)),
          pl.BlockSpec((D, D), lambda i: (0, 0)),
          pl.BlockSpec((1, D), lambda i: (0, 0)),
          pl.BlockSpec((D, D), lambda i: (0, 0)),
          pl.BlockSpec((1, D), lambda i: (0, 0)),
      ],
      out_specs=pl.BlockSpec((BLK, D), lambda i: (i, 0)),
      out_shape=jax.ShapeDtypeStruct((N_NODES, D), jnp.float32),
  )(eps, nh, parts, W1, b1.reshape(1, D), W2, b2.reshape(1, D))


@jax.jit
def _gin(nh, edge_index, W1, b1, W2, b2, eps):
  ei = edge_index.astype(jnp.int32)
  src2 = (ei[0] * 2).reshape(NS, NCHUNK, CHUNK)
  dst = ei[1].reshape(NS, NCHUNK, CHUNK)
  nh2 = nh.reshape(2 * N_NODES, DH)
  parts = _sc_scatter(nh2, src2, dst)
  return _tc_mlp(nh, parts, W1, b1, W2, b2, eps)


def kernel(nh, eh, edge_index, W1, b1, W2, b2, eps):
  return (_gin(nh, edge_index, W1, b1, W2, b2, eps), eh)
